# Initial kernel scaffold; baseline (speedup 1.0000x reference)
import jax
import jax.numpy as jnp
from jax.experimental import pallas as pl


def kernel(shape, color, pose, instance_id, t, pad, shape_table, color_table, instance_table, temporal_table, W_pose, b_pose, pe_g, pe_b, g_s, b_s, g_c, b_c, g_p, b_p, g_i, b_i, g_sum, b_sum):
    raise NotImplementedError("write your pallas kernel here")


# speedup vs baseline: 3.1967x; 3.1967x over previous
import functools

import jax
import jax.numpy as jnp
from jax import lax
from jax.experimental import pallas as pl
from jax.experimental.pallas import tpu as pltpu
from jax.experimental.pallas import tpu_sc as plsc

C = 64
TSCALE = 0.005

B, L = 1024, 200
N = B * L
VROWS = 1024
NTBL = 4

_NC = 2
_NS = 16
_NW = _NC * _NS
_T = 128
_TOKW = N // _NW
_CHUNKS = _TOKW // _T


def _ln_rows(x, g, b, eps=1e-5):
    m = jnp.mean(x, axis=-1, keepdims=True)
    v = jnp.mean((x - m) ** 2, axis=-1, keepdims=True)
    return (x - m) / jnp.sqrt(v + eps) * g + b


def _table_norm_body(tbl_ref, g_ref, b_ref, out_ref):
    p = pl.program_id(0)
    x = tbl_ref[0]
    ln = _ln_rows(x, g_ref[0], b_ref[0])
    keep = (p == NTBL - 1).astype(jnp.float32)
    out_ref[0] = x * keep + ln * (1.0 - keep)


def _normalize_tables(stacked, gs, bs):
    return pl.pallas_call(
        _table_norm_body,
        grid=(NTBL,),
        in_specs=[
            pl.BlockSpec((1, VROWS, C), lambda p: (p, 0, 0)),
            pl.BlockSpec((1, 1, C), lambda p: (p, 0, 0)),
            pl.BlockSpec((1, 1, C), lambda p: (p, 0, 0)),
        ],
        out_specs=pl.BlockSpec((1, VROWS, C), lambda p: (p, 0, 0)),
        out_shape=jax.ShapeDtypeStruct((NTBL, VROWS, C), jnp.float32),
    )(stacked, gs.reshape(NTBL, 1, C), bs.reshape(NTBL, 1, C))


def _sc_gather_sum_body(tbl_hbm, sidx_hbm, cidx_hbm, iidx_hbm, tidx_hbm,
                        out_hbm, idx_v, rows_v, sem):
    wid = lax.axis_index("s") * _NC + lax.axis_index("c")

    def chunk(ci, carry):
        base = wid * _TOKW + ci * _T
        pltpu.sync_copy(sidx_hbm.at[pl.ds(base, _T)], idx_v.at[0])
        pltpu.sync_copy(cidx_hbm.at[pl.ds(base, _T)], idx_v.at[1])
        pltpu.sync_copy(iidx_hbm.at[pl.ds(base, _T)], idx_v.at[2])
        pltpu.sync_copy(tidx_hbm.at[pl.ds(base, _T)], idx_v.at[3])
        for r in range(1, NTBL):
            for g in range(_T // 16):
                sl = pl.ds(g * 16, 16)
                idx_v[r, sl] = idx_v[r, sl] + (r * VROWS)
        for r in range(NTBL):
            pltpu.async_copy(tbl_hbm.at[idx_v.at[r]],
                             rows_v.at[pl.ds(r * _T, _T)], sem).wait()

        def tok(tk, c2):
            for g in range(C // 16):
                sl = pl.ds(g * 16, 16)
                s0 = rows_v[tk, sl] + rows_v[tk + _T, sl]
                s1 = rows_v[tk + 2 * _T, sl] + rows_v[tk + 3 * _T, sl]
                rows_v[tk, sl] = s0 + s1
            return c2

        lax.fori_loop(0, _T, tok, 0, unroll=2)
        pltpu.sync_copy(rows_v.at[pl.ds(0, _T)], out_hbm.at[pl.ds(base, _T)])
        return carry

    lax.fori_loop(0, _CHUNKS, chunk, 0)


def _sc_gather_sum(tbl, sidx, cidx, iidx, tidx):
    mesh = plsc.VectorSubcoreMesh(core_axis_name="c", subcore_axis_name="s")
    run = functools.partial(
        pl.kernel,
        mesh=mesh,
        compiler_params=pltpu.CompilerParams(use_tc_tiling_on_sc=False),
        out_type=jax.ShapeDtypeStruct((N, C), jnp.float32),
        scratch_types=[
            pltpu.VMEM((NTBL, _T), jnp.int32),
            pltpu.VMEM((NTBL * _T, C), jnp.float32),
            pltpu.SemaphoreType.DMA,
        ],
    )(_sc_gather_sum_body)
    return run(tbl, sidx, cidx, iidx, tidx)


_R = 512


def _combine_body(esum_ref, pose_ref, w_ref, bp_ref, peg_ref, peb_ref,
                  gp_ref, bpn_ref, gs_ref, bs_ref, out_ref):
    px = pose_ref[...]
    h = jnp.dot(px, w_ref[...], preferred_element_type=jnp.float32) + bp_ref[0]
    h = _ln_rows(h, peg_ref[0], peb_ref[0])
    h = _ln_rows(h, gp_ref[0], bpn_ref[0])
    x = esum_ref[...] + h
    out_ref[...] = _ln_rows(x, gs_ref[0], bs_ref[0])


def _combine(esum, pose16, w16, b_pose, pe_g, pe_b, g_p, b_p, g_sum, b_sum):
    vec = lambda: pl.BlockSpec((1, C), lambda i: (0, 0))
    return pl.pallas_call(
        _combine_body,
        grid=(N // _R,),
        in_specs=[
            pl.BlockSpec((_R, C), lambda i: (i, 0)),
            pl.BlockSpec((_R, 16), lambda i: (i, 0)),
            pl.BlockSpec((16, C), lambda i: (0, 0)),
            vec(), vec(), vec(), vec(), vec(), vec(), vec(),
        ],
        out_specs=pl.BlockSpec((_R, C), lambda i: (i, 0)),
        out_shape=jax.ShapeDtypeStruct((N, C), jnp.float32),
    )(esum, pose16, w16, b_pose.reshape(1, C), pe_g.reshape(1, C),
      pe_b.reshape(1, C), g_p.reshape(1, C), b_p.reshape(1, C),
      g_sum.reshape(1, C), b_sum.reshape(1, C))


def kernel(shape, color, pose, instance_id, t, pad,
           shape_table, color_table, instance_table, temporal_table,
           W_pose, b_pose, pe_g, pe_b,
           g_s, b_s, g_c, b_c, g_p, b_p, g_i, b_i, g_sum, b_sum):
    f32 = jnp.float32
    pad_rows = lambda tb: jnp.pad(tb, ((0, VROWS - tb.shape[0]), (0, 0)))
    stacked = jnp.stack([
        pad_rows(shape_table.astype(f32)),
        pad_rows(color_table.astype(f32)),
        pad_rows(instance_table.astype(f32)),
        temporal_table.astype(f32),
    ])
    gs = jnp.stack([g_s, g_c, g_i, jnp.ones_like(g_s)]).astype(f32)
    bs = jnp.stack([b_s, b_c, b_i, jnp.zeros_like(b_s)]).astype(f32)
    tbl = _normalize_tables(stacked, gs, bs).reshape(NTBL * VROWS, C)

    i32 = jnp.int32
    sidx = shape.reshape(N).astype(i32)
    cidx = color.reshape(N).astype(i32)
    iidx = instance_id.reshape(N).astype(i32)
    tidx = t.reshape(N).astype(i32)
    esum = _sc_gather_sum(tbl, sidx, cidx, iidx, tidx)

    scale12 = jnp.tile(jnp.array([1.0, 1.0, 1.0, TSCALE], dtype=f32), 3)
    w16 = jnp.concatenate(
        [W_pose.astype(f32) * scale12[:, None], jnp.zeros((4, C), dtype=f32)])
    pose16 = pose.astype(f32).reshape(N, 16)

    x = _combine(esum, pose16, w16, b_pose, pe_g, pe_b, g_p, b_p,
                 g_sum, b_sum).reshape(B, L, C)
    return (x, t, pad)


# speedup vs baseline: 4.2876x; 1.3413x over previous
import functools

import jax
import jax.numpy as jnp
from jax import lax
from jax.experimental import pallas as pl
from jax.experimental.pallas import tpu as pltpu
from jax.experimental.pallas import tpu_sc as plsc

C = 64
TSCALE = 0.005

B, L = 1024, 200
N = B * L
VROWS = 1024
NTBL = 4

_NC = 2
_NS = 16
_NW = _NC * _NS
_T = 128
_TOKW = N // _NW
_CHUNKS = _TOKW // _T


def _ln_rows(x, g, b, eps=1e-5):
    m = jnp.mean(x, axis=-1, keepdims=True)
    v = jnp.mean((x - m) ** 2, axis=-1, keepdims=True)
    return (x - m) / jnp.sqrt(v + eps) * g + b


def _table_norm_body(tbl_ref, g_ref, b_ref, out_ref):
    p = pl.program_id(0)
    x = tbl_ref[0]
    ln = _ln_rows(x, g_ref[0], b_ref[0])
    keep = (p == NTBL - 1).astype(jnp.float32)
    out_ref[0] = x * keep + ln * (1.0 - keep)


def _normalize_tables(stacked, gs, bs):
    return pl.pallas_call(
        _table_norm_body,
        grid=(NTBL,),
        in_specs=[
            pl.BlockSpec((1, VROWS, C), lambda p: (p, 0, 0)),
            pl.BlockSpec((1, 1, C), lambda p: (p, 0, 0)),
            pl.BlockSpec((1, 1, C), lambda p: (p, 0, 0)),
        ],
        out_specs=pl.BlockSpec((1, VROWS, C), lambda p: (p, 0, 0)),
        out_shape=jax.ShapeDtypeStruct((NTBL, VROWS, C), jnp.float32),
    )(stacked, gs.reshape(NTBL, 1, C), bs.reshape(NTBL, 1, C))


def _sc_gather_sum_body(tbl_hbm, sidx_hbm, cidx_hbm, iidx_hbm, tidx_hbm,
                        out_hbm, tbl_sh, idx0, idx1, rows0, rows1,
                        si0, si1, sg0, sg1, so0, so1):
    wid = lax.axis_index("s") * _NC + lax.axis_index("c")
    sid = lax.axis_index("s")

    @pl.when(sid == 0)
    def _():
        pltpu.sync_copy(tbl_hbm, tbl_sh)
    plsc.subcore_barrier()

    idx_hbms = (sidx_hbm, cidx_hbm, iidx_hbm, tidx_hbm)

    def fire_idx(base, idx_v, sem):
        return [pltpu.async_copy(h.at[pl.ds(base, _T)], idx_v.at[r], sem)
                for r, h in enumerate(idx_hbms)]

    def offset_idx(idx_v):
        for r in range(1, NTBL):
            for g in range(_T // 16):
                sl = pl.ds(g * 16, 16)
                idx_v[r, sl] = idx_v[r, sl] + (r * VROWS)

    def fire_gather(idx_v, rows_v, sem):
        return [pltpu.async_copy(tbl_sh.at[idx_v.at[r]],
                                 rows_v.at[pl.ds(r * _T, _T)], sem)
                for r in range(NTBL)]

    def sum_rows(rv):
        def tok(tk, c2):
            for g in range(C // 16):
                sl = pl.ds(g * 16, 16)
                s0 = rv[tk, sl] + rv[tk + _T, sl]
                s1 = rv[tk + 2 * _T, sl] + rv[tk + 3 * _T, sl]
                rv[tk, sl] = s0 + s1
            return c2

        lax.fori_loop(0, _T, tok, 0, unroll=2)

    def outer(g2, carry):
        base0 = wid * _TOKW + (2 * g2) * _T
        base1 = base0 + _T
        hi0 = fire_idx(base0, idx0, si0)
        hi1 = fire_idx(base1, idx1, si1)
        for h in hi0:
            h.wait()
        offset_idx(idx0)
        hg0 = fire_gather(idx0, rows0, sg0)
        for h in hi1:
            h.wait()
        offset_idx(idx1)
        hg1 = fire_gather(idx1, rows1, sg1)
        for h in hg0:
            h.wait()
        sum_rows(rows0)
        ho0 = pltpu.async_copy(rows0.at[pl.ds(0, _T)],
                               out_hbm.at[pl.ds(base0, _T)], so0)
        for h in hg1:
            h.wait()
        sum_rows(rows1)
        ho1 = pltpu.async_copy(rows1.at[pl.ds(0, _T)],
                               out_hbm.at[pl.ds(base1, _T)], so1)
        ho0.wait()
        ho1.wait()
        return carry

    lax.fori_loop(0, _CHUNKS // 2, outer, 0)


def _sc_gather_sum(tbl, sidx, cidx, iidx, tidx):
    mesh = plsc.VectorSubcoreMesh(core_axis_name="c", subcore_axis_name="s")
    run = functools.partial(
        pl.kernel,
        mesh=mesh,
        compiler_params=pltpu.CompilerParams(use_tc_tiling_on_sc=False),
        out_type=jax.ShapeDtypeStruct((N, C), jnp.float32),
        scratch_types=[
            pltpu.VMEM_SHARED((NTBL * VROWS, C), jnp.float32),
            pltpu.VMEM((NTBL, _T), jnp.int32),
            pltpu.VMEM((NTBL, _T), jnp.int32),
            pltpu.VMEM((NTBL * _T, C), jnp.float32),
            pltpu.VMEM((NTBL * _T, C), jnp.float32),
            pltpu.SemaphoreType.DMA,
            pltpu.SemaphoreType.DMA,
            pltpu.SemaphoreType.DMA,
            pltpu.SemaphoreType.DMA,
            pltpu.SemaphoreType.DMA,
            pltpu.SemaphoreType.DMA,
        ],
    )(_sc_gather_sum_body)
    return run(tbl, sidx, cidx, iidx, tidx)


_R = 512


def _combine_body(esum_ref, pose_ref, w_ref, bp_ref, peg_ref, peb_ref,
                  gp_ref, bpn_ref, gs_ref, bs_ref, out_ref):
    px = pose_ref[...]
    h = jnp.dot(px, w_ref[...], preferred_element_type=jnp.float32) + bp_ref[0]
    h = _ln_rows(h, peg_ref[0], peb_ref[0])
    h = _ln_rows(h, gp_ref[0], bpn_ref[0])
    x = esum_ref[...] + h
    out_ref[...] = _ln_rows(x, gs_ref[0], bs_ref[0])


def _combine(esum, pose16, w16, b_pose, pe_g, pe_b, g_p, b_p, g_sum, b_sum):
    vec = lambda: pl.BlockSpec((1, C), lambda i: (0, 0))
    return pl.pallas_call(
        _combine_body,
        grid=(N // _R,),
        in_specs=[
            pl.BlockSpec((_R, C), lambda i: (i, 0)),
            pl.BlockSpec((_R, 16), lambda i: (i, 0)),
            pl.BlockSpec((16, C), lambda i: (0, 0)),
            vec(), vec(), vec(), vec(), vec(), vec(), vec(),
        ],
        out_specs=pl.BlockSpec((_R, C), lambda i: (i, 0)),
        out_shape=jax.ShapeDtypeStruct((N, C), jnp.float32),
    )(esum, pose16, w16, b_pose.reshape(1, C), pe_g.reshape(1, C),
      pe_b.reshape(1, C), g_p.reshape(1, C), b_p.reshape(1, C),
      g_sum.reshape(1, C), b_sum.reshape(1, C))


def kernel(shape, color, pose, instance_id, t, pad,
           shape_table, color_table, instance_table, temporal_table,
           W_pose, b_pose, pe_g, pe_b,
           g_s, b_s, g_c, b_c, g_p, b_p, g_i, b_i, g_sum, b_sum):
    f32 = jnp.float32
    pad_rows = lambda tb: jnp.pad(tb, ((0, VROWS - tb.shape[0]), (0, 0)))
    stacked = jnp.stack([
        pad_rows(shape_table.astype(f32)),
        pad_rows(color_table.astype(f32)),
        pad_rows(instance_table.astype(f32)),
        temporal_table.astype(f32),
    ])
    gs = jnp.stack([g_s, g_c, g_i, jnp.ones_like(g_s)]).astype(f32)
    bs = jnp.stack([b_s, b_c, b_i, jnp.zeros_like(b_s)]).astype(f32)
    tbl = _normalize_tables(stacked, gs, bs).reshape(NTBL * VROWS, C)

    i32 = jnp.int32
    sidx = shape.reshape(N).astype(i32)
    cidx = color.reshape(N).astype(i32)
    iidx = instance_id.reshape(N).astype(i32)
    tidx = t.reshape(N).astype(i32)
    esum = _sc_gather_sum(tbl, sidx, cidx, iidx, tidx)

    scale12 = jnp.tile(jnp.array([1.0, 1.0, 1.0, TSCALE], dtype=f32), 3)
    w16 = jnp.concatenate(
        [W_pose.astype(f32) * scale12[:, None], jnp.zeros((4, C), dtype=f32)])
    pose16 = pose.astype(f32).reshape(N, 16)

    x = _combine(esum, pose16, w16, b_pose, pe_g, pe_b, g_p, b_p,
                 g_sum, b_sum).reshape(B, L, C)
    return (x, t, pad)
